# baseline (speedup 1.0000x reference)
import dataclasses
import functools

import jax
import jax.numpy as jnp
import numpy as np
from jax import lax
from jax.experimental import pallas as pl
from jax.experimental.pallas import tpu as pltpu
from jax.experimental.pallas import tpu_sc as plsc

B = 16384
E = 64
D = 4096
TEMPER = 0.5

NC = 2
NS = 16
NW = NC * NS
L = 16
R = 128
U = 4
CW = D // NW
NCHUNK = B // R


def _routing_body(prob_ref, gum_ref, scal_ref, gs_ref, maxpos_ref, sel_ref,
                  den_ref):
    prob = prob_ref[...]
    m1 = jnp.max(prob, axis=1, keepdims=True)
    e1 = jnp.exp(prob - m1)
    sm1 = e1 / jnp.sum(e1, axis=1, keepdims=True)
    z = (sm1 * scal_ref[0, 0] + gum_ref[...]) / TEMPER
    m2 = jnp.max(z, axis=1, keepdims=True)
    e2 = jnp.exp(z - m2)
    gs = e2 / jnp.sum(e2, axis=1, keepdims=True)
    gs_ref[...] = gs
    mx = jnp.max(gs, axis=1, keepdims=True)
    col = lax.broadcasted_iota(jnp.int32, gs.shape, 1)
    idx = jnp.min(jnp.where(gs == mx, col, E), axis=1, keepdims=True)
    maxpos_ref[...] = idx
    sel_ref[...] = mx
    onehot = col == idx
    den_ref[...] = jnp.sum(jnp.where(onehot, jnp.broadcast_to(mx, gs.shape),
                                     0.0), axis=0, keepdims=True)


def _routing(probability, gumbel, scal):
    return pl.pallas_call(
        _routing_body,
        out_shape=[
            jax.ShapeDtypeStruct((B, E), jnp.float32),
            jax.ShapeDtypeStruct((B, 1), jnp.int32),
            jax.ShapeDtypeStruct((B, 1), jnp.float32),
            jax.ShapeDtypeStruct((1, E), jnp.float32),
        ],
    )(probability, gumbel, scal)


def _sc_segment_sum(stateLis, maxpos, sel, den):
    mesh = plsc.VectorSubcoreMesh(core_axis_name="c", subcore_axis_name="s")
    cp = pltpu.CompilerParams()
    if "needs_layout_passes" in pltpu.CompilerParams.__dataclass_fields__:
        cp = dataclasses.replace(cp, needs_layout_passes=False)

    @functools.partial(
        pl.kernel,
        out_type=jax.ShapeDtypeStruct((E, D), jnp.float32),
        mesh=mesh,
        compiler_params=cp,
        scratch_types=[
            pltpu.VMEM((R, CW), jnp.float32),
            pltpu.VMEM((R, CW), jnp.float32),
            pltpu.VMEM((R,), jnp.int32),
            pltpu.VMEM((R,), jnp.int32),
            pltpu.VMEM((R,), jnp.float32),
            pltpu.VMEM((R,), jnp.float32),
            pltpu.VMEM((E,), jnp.float32),
            pltpu.VMEM((E, CW), jnp.float32),
            pltpu.VMEM((E, CW), jnp.float32),
            pltpu.VMEM((E, CW), jnp.float32),
            pltpu.VMEM((E, CW), jnp.float32),
            pltpu.SemaphoreType.DMA,
            pltpu.SemaphoreType.DMA,
        ],
    )
    def kern(x_hbm, idx_hbm, sel_hbm, den_hbm, out_hbm,
             xb0, xb1, ib0, ib1, sb0, sb1, denb, acc0, acc1, acc2, acc3,
             sem0, sem1):
        accs = (acc0, acc1, acc2, acc3)
        c = lax.axis_index("c")
        s = lax.axis_index("s")
        w = s * NC + c
        col0 = w * CW

        zero16 = jnp.zeros((L,), jnp.float32)

        @pl.loop(0, E)
        def _(e):
            for a in accs:
                for g in range(CW // L):
                    a[e, pl.ds(g * L, L)] = zero16

        def start_load(k, xb, ib, sb, sem):
            r0 = k * R
            pltpu.async_copy(x_hbm.at[pl.ds(r0, R), pl.ds(col0, CW)], xb, sem)
            pltpu.async_copy(idx_hbm.at[pl.ds(r0, R)], ib, sem)
            pltpu.async_copy(sel_hbm.at[pl.ds(r0, R)], sb, sem)

        def wait_load(k, xb, ib, sb, sem):
            r0 = k * R
            pltpu.make_async_copy(
                x_hbm.at[pl.ds(r0, R), pl.ds(col0, CW)], xb, sem).wait()
            pltpu.make_async_copy(idx_hbm.at[pl.ds(r0, R)], ib, sem).wait()
            pltpu.make_async_copy(sel_hbm.at[pl.ds(r0, R)], sb, sem).wait()

        def process(xb, ib, sb):
            @pl.loop(0, R, step=U)
            def _(r):
                evs = []
                svs = []
                for i in range(U):
                    rv = lax.broadcast(r + i, (L,))
                    evs.append(plsc.load_gather(ib, [rv]))
                    svs.append(plsc.load_gather(sb, [rv]))
                for g in range(CW // L):
                    sl = pl.ds(g * L, L)
                    colv = lax.iota(jnp.int32, L) + g * L
                    xs = [xb[r + i, sl] for i in range(U)]
                    vs = [xs[i] * svs[i] for i in range(U)]
                    for i in range(U):
                        plsc.addupdate_scatter(accs[i], [evs[i], colv], vs[i])

        start_load(0, xb0, ib0, sb0, sem0)

        @pl.loop(0, NCHUNK, step=2)
        def _(k):
            start_load(k + 1, xb1, ib1, sb1, sem1)
            wait_load(k, xb0, ib0, sb0, sem0)
            process(xb0, ib0, sb0)

            @pl.when(k + 2 < NCHUNK)
            def _():
                start_load(k + 2, xb0, ib0, sb0, sem0)

            wait_load(k + 1, xb1, ib1, sb1, sem1)
            process(xb1, ib1, sb1)

        pltpu.sync_copy(den_hbm, denb)
        for g in range(E // L):
            sl = pl.ds(g * L, L)
            dv = denb[sl]
            denb[sl] = jnp.where(dv > 0.0, 1.0 / dv, 1.0)

        @pl.loop(0, E)
        def _(e):
            evv = lax.broadcast(e, (L,))
            iv = plsc.load_gather(denb, [evv])
            for g in range(CW // L):
                sl = pl.ds(g * L, L)
                acc0[e, sl] = ((acc0[e, sl] + acc1[e, sl])
                               + (acc2[e, sl] + acc3[e, sl])) * iv

        pltpu.sync_copy(acc0, out_hbm.at[:, pl.ds(col0, CW)])

    return kern(stateLis, maxpos, sel, den)


@functools.lru_cache(maxsize=1)
def _gumbel_host():
    with jax.ensure_compile_time_eval():
        gkey = jax.random.fold_in(jax.random.key(0), 7)
        return np.asarray(jax.random.gumbel(gkey, (B, E), dtype=jnp.float32))


def kernel(stateLis, probability, LastCliOnSerDic, scalar):
    gumbel = jnp.asarray(_gumbel_host())
    scal = jnp.asarray(scalar, jnp.float32).reshape(1, 1)

    gs, maxpos2, sel2, den2 = _routing(probability, gumbel, scal)
    maxpos = maxpos2.reshape(B)
    sel = sel2.reshape(B)
    den = den2.reshape(E)

    agg = _sc_segment_sum(stateLis, maxpos, sel, den)
    return (gs, agg, maxpos)


# speedup vs baseline: 1.1995x; 1.1995x over previous
import dataclasses
import functools

import jax
import jax.numpy as jnp
import numpy as np
from jax import lax
from jax.experimental import pallas as pl
from jax.experimental.pallas import tpu as pltpu
from jax.experimental.pallas import tpu_sc as plsc

B = 16384
E = 64
D = 4096
TEMPER = 0.5

NC = 2
NS = 16
NW = NC * NS
L = 16
R = 128
U = 8
CW = D // NW
NCHUNK = B // R


def _routing_body(prob_ref, gum_ref, scal_ref, gs_ref, maxpos_ref, sel_ref,
                  den_ref):
    prob = prob_ref[...]
    m1 = jnp.max(prob, axis=1, keepdims=True)
    e1 = jnp.exp(prob - m1)
    sm1 = e1 / jnp.sum(e1, axis=1, keepdims=True)
    z = (sm1 * scal_ref[0, 0] + gum_ref[...]) / TEMPER
    m2 = jnp.max(z, axis=1, keepdims=True)
    e2 = jnp.exp(z - m2)
    gs = e2 / jnp.sum(e2, axis=1, keepdims=True)
    gs_ref[...] = gs
    mx = jnp.max(gs, axis=1, keepdims=True)
    col = lax.broadcasted_iota(jnp.int32, gs.shape, 1)
    idx = jnp.min(jnp.where(gs == mx, col, E), axis=1, keepdims=True)
    maxpos_ref[...] = idx
    sel_ref[...] = mx
    onehot = col == idx
    den_ref[...] = jnp.sum(jnp.where(onehot, jnp.broadcast_to(mx, gs.shape),
                                     0.0), axis=0, keepdims=True)


def _routing(probability, gumbel, scal):
    return pl.pallas_call(
        _routing_body,
        out_shape=[
            jax.ShapeDtypeStruct((B, E), jnp.float32),
            jax.ShapeDtypeStruct((B, 1), jnp.int32),
            jax.ShapeDtypeStruct((B, 1), jnp.float32),
            jax.ShapeDtypeStruct((1, E), jnp.float32),
        ],
    )(probability, gumbel, scal)


def _sc_segment_sum(stateLis, maxpos, sel, den):
    mesh = plsc.VectorSubcoreMesh(core_axis_name="c", subcore_axis_name="s")
    cp = pltpu.CompilerParams()
    if "needs_layout_passes" in pltpu.CompilerParams.__dataclass_fields__:
        cp = dataclasses.replace(cp, needs_layout_passes=False)

    @functools.partial(
        pl.kernel,
        out_type=jax.ShapeDtypeStruct((E, D), jnp.float32),
        mesh=mesh,
        compiler_params=cp,
        scratch_types=[
            pltpu.VMEM((R, CW), jnp.float32),
            pltpu.VMEM((R, CW), jnp.float32),
            pltpu.VMEM((R,), jnp.int32),
            pltpu.VMEM((R,), jnp.int32),
            pltpu.VMEM((R,), jnp.float32),
            pltpu.VMEM((R,), jnp.float32),
            pltpu.VMEM((E,), jnp.float32),
        ] + [pltpu.VMEM((E, CW), jnp.float32) for _ in range(U)] + [
            pltpu.SemaphoreType.DMA,
            pltpu.SemaphoreType.DMA,
        ],
    )
    def kern(x_hbm, idx_hbm, sel_hbm, den_hbm, out_hbm,
             xb0, xb1, ib0, ib1, sb0, sb1, denb, *rest):
        accs = rest[:U]
        sem0, sem1 = rest[U], rest[U + 1]
        c = lax.axis_index("c")
        s = lax.axis_index("s")
        w = s * NC + c
        col0 = w * CW

        zero16 = jnp.zeros((L,), jnp.float32)

        @pl.loop(0, E)
        def _(e):
            for a in accs:
                for g in range(CW // L):
                    a[e, pl.ds(g * L, L)] = zero16

        def start_load(k, xb, ib, sb, sem):
            r0 = k * R
            pltpu.async_copy(x_hbm.at[pl.ds(r0, R), pl.ds(col0, CW)], xb, sem)
            pltpu.async_copy(idx_hbm.at[pl.ds(r0, R)], ib, sem)
            pltpu.async_copy(sel_hbm.at[pl.ds(r0, R)], sb, sem)

        def wait_load(k, xb, ib, sb, sem):
            r0 = k * R
            pltpu.make_async_copy(
                x_hbm.at[pl.ds(r0, R), pl.ds(col0, CW)], xb, sem).wait()
            pltpu.make_async_copy(idx_hbm.at[pl.ds(r0, R)], ib, sem).wait()
            pltpu.make_async_copy(sel_hbm.at[pl.ds(r0, R)], sb, sem).wait()

        def process(xb, ib, sb):
            @pl.loop(0, R, step=U)
            def _(r):
                evs = []
                svs = []
                for i in range(U):
                    rv = lax.broadcast(r + i, (L,))
                    evs.append(plsc.load_gather(ib, [rv]))
                    svs.append(plsc.load_gather(sb, [rv]))
                for g in range(CW // L):
                    sl = pl.ds(g * L, L)
                    colv = lax.iota(jnp.int32, L) + g * L
                    xs = [xb[r + i, sl] for i in range(U)]
                    vs = [xs[i] * svs[i] for i in range(U)]
                    for i in range(U):
                        plsc.addupdate_scatter(accs[i], [evs[i], colv], vs[i])

        start_load(0, xb0, ib0, sb0, sem0)

        @pl.loop(0, NCHUNK, step=2)
        def _(k):
            start_load(k + 1, xb1, ib1, sb1, sem1)
            wait_load(k, xb0, ib0, sb0, sem0)
            process(xb0, ib0, sb0)

            @pl.when(k + 2 < NCHUNK)
            def _():
                start_load(k + 2, xb0, ib0, sb0, sem0)

            wait_load(k + 1, xb1, ib1, sb1, sem1)
            process(xb1, ib1, sb1)

        pltpu.sync_copy(den_hbm, denb)
        for g in range(E // L):
            sl = pl.ds(g * L, L)
            dv = denb[sl]
            denb[sl] = jnp.where(dv > 0.0, 1.0 / dv, 1.0)

        @pl.loop(0, E)
        def _(e):
            evv = lax.broadcast(e, (L,))
            iv = plsc.load_gather(denb, [evv])
            for g in range(CW // L):
                sl = pl.ds(g * L, L)
                parts = [a[e, sl] for a in accs]
                while len(parts) > 1:
                    parts = [parts[i] + parts[i + 1]
                             for i in range(0, len(parts), 2)]
                accs[0][e, sl] = parts[0] * iv

        pltpu.sync_copy(accs[0], out_hbm.at[:, pl.ds(col0, CW)])

    return kern(stateLis, maxpos, sel, den)


@functools.lru_cache(maxsize=1)
def _gumbel_host():
    with jax.ensure_compile_time_eval():
        gkey = jax.random.fold_in(jax.random.key(0), 7)
        return np.asarray(jax.random.gumbel(gkey, (B, E), dtype=jnp.float32))


def kernel(stateLis, probability, LastCliOnSerDic, scalar):
    gumbel = jnp.asarray(_gumbel_host())
    scal = jnp.asarray(scalar, jnp.float32).reshape(1, 1)

    gs, maxpos2, sel2, den2 = _routing(probability, gumbel, scal)
    maxpos = maxpos2.reshape(B)
    sel = sel2.reshape(B)
    den = den2.reshape(E)

    agg = _sc_segment_sum(stateLis, maxpos, sel, den)
    return (gs, agg, maxpos)


# speedup vs baseline: 1.9070x; 1.5898x over previous
import dataclasses
import functools

import jax
import jax.numpy as jnp
import numpy as np
from jax import lax
from jax.experimental import pallas as pl
from jax.experimental.pallas import tpu as pltpu
from jax.experimental.pallas import tpu_sc as plsc

B = 16384
E = 64
D = 4096
TEMPER = 0.5

NC = 2
NS = 16
NW = NC * NS
L = 16
R = 128
U = 8
CW = D // NW
BSC = 6144
BTC = B - BSC
BK = 512
NCHUNK = BSC // R


def _routing_body(prob_ref, gum_ref, scal_ref, gs_ref, maxpos_ref, sel_ref,
                  w_ref, den_ref):
    blk = pl.program_id(0)
    prob = prob_ref[...]
    m1 = jnp.max(prob, axis=1, keepdims=True)
    e1 = jnp.exp(prob - m1)
    sm1 = e1 / jnp.sum(e1, axis=1, keepdims=True)
    z = (sm1 * scal_ref[0, 0] + gum_ref[...]) / TEMPER
    m2 = jnp.max(z, axis=1, keepdims=True)
    e2 = jnp.exp(z - m2)
    gs = e2 / jnp.sum(e2, axis=1, keepdims=True)
    gs_ref[...] = gs
    mx = jnp.max(gs, axis=1, keepdims=True)
    col = lax.broadcasted_iota(jnp.int32, gs.shape, 1)
    idx = jnp.min(jnp.where(gs == mx, col, E), axis=1, keepdims=True)
    maxpos_ref[...] = idx
    sel_ref[...] = mx
    onehot = col == idx
    w = jnp.where(onehot, jnp.broadcast_to(mx, gs.shape), 0.0)
    w_ref[...] = w

    @pl.when(blk == 0)
    def _():
        den_ref[...] = jnp.zeros_like(den_ref)

    den_ref[...] += jnp.sum(w, axis=0, keepdims=True)


RB = 2048


def _routing(probability, gumbel, scal):
    return pl.pallas_call(
        _routing_body,
        grid=(B // RB,),
        in_specs=[
            pl.BlockSpec((RB, E), lambda i: (i, 0)),
            pl.BlockSpec((RB, E), lambda i: (i, 0)),
            pl.BlockSpec((1, 1), lambda i: (0, 0)),
        ],
        out_specs=[
            pl.BlockSpec((RB, E), lambda i: (i, 0)),
            pl.BlockSpec((RB, 1), lambda i: (i, 0)),
            pl.BlockSpec((RB, 1), lambda i: (i, 0)),
            pl.BlockSpec((RB, E), lambda i: (i, 0)),
            pl.BlockSpec((1, E), lambda i: (0, 0)),
        ],
        out_shape=[
            jax.ShapeDtypeStruct((B, E), jnp.float32),
            jax.ShapeDtypeStruct((B, 1), jnp.int32),
            jax.ShapeDtypeStruct((B, 1), jnp.float32),
            jax.ShapeDtypeStruct((B, E), jnp.float32),
            jax.ShapeDtypeStruct((1, E), jnp.float32),
        ],
    )(probability, gumbel, scal)


def _sc_segment_sum(stateLis, maxpos, sel):
    mesh = plsc.VectorSubcoreMesh(core_axis_name="c", subcore_axis_name="s")
    cp = pltpu.CompilerParams()
    if "needs_layout_passes" in pltpu.CompilerParams.__dataclass_fields__:
        cp = dataclasses.replace(cp, needs_layout_passes=False)

    @functools.partial(
        pl.kernel,
        out_type=jax.ShapeDtypeStruct((E, D), jnp.float32),
        mesh=mesh,
        compiler_params=cp,
        scratch_types=[
            pltpu.VMEM((R, CW), jnp.float32),
            pltpu.VMEM((R, CW), jnp.float32),
            pltpu.VMEM((R,), jnp.int32),
            pltpu.VMEM((R,), jnp.int32),
            pltpu.VMEM((R,), jnp.float32),
            pltpu.VMEM((R,), jnp.float32),
        ] + [pltpu.VMEM((E, CW), jnp.float32) for _ in range(U)] + [
            pltpu.SemaphoreType.DMA,
            pltpu.SemaphoreType.DMA,
        ],
    )
    def kern(x_hbm, idx_hbm, sel_hbm, out_hbm,
             xb0, xb1, ib0, ib1, sb0, sb1, *rest):
        accs = rest[:U]
        sem0, sem1 = rest[U], rest[U + 1]
        c = lax.axis_index("c")
        s = lax.axis_index("s")
        w = s * NC + c
        col0 = w * CW

        zero16 = jnp.zeros((L,), jnp.float32)

        @pl.loop(0, E)
        def _(e):
            for a in accs:
                for g in range(CW // L):
                    a[e, pl.ds(g * L, L)] = zero16

        def start_load(k, xb, ib, sb, sem):
            r0 = k * R
            pltpu.async_copy(x_hbm.at[pl.ds(r0, R), pl.ds(col0, CW)], xb, sem)
            pltpu.async_copy(idx_hbm.at[pl.ds(r0, R)], ib, sem)
            pltpu.async_copy(sel_hbm.at[pl.ds(r0, R)], sb, sem)

        def wait_load(k, xb, ib, sb, sem):
            r0 = k * R
            pltpu.make_async_copy(
                x_hbm.at[pl.ds(r0, R), pl.ds(col0, CW)], xb, sem).wait()
            pltpu.make_async_copy(idx_hbm.at[pl.ds(r0, R)], ib, sem).wait()
            pltpu.make_async_copy(sel_hbm.at[pl.ds(r0, R)], sb, sem).wait()

        def process(xb, ib, sb):
            @pl.loop(0, R, step=U)
            def _(r):
                evs = []
                svs = []
                for i in range(U):
                    rv = lax.broadcast(r + i, (L,))
                    evs.append(plsc.load_gather(ib, [rv]))
                    svs.append(plsc.load_gather(sb, [rv]))
                for g in range(CW // L):
                    sl = pl.ds(g * L, L)
                    colv = lax.iota(jnp.int32, L) + g * L
                    xs = [xb[r + i, sl] for i in range(U)]
                    vs = [xs[i] * svs[i] for i in range(U)]
                    for i in range(U):
                        plsc.addupdate_scatter(accs[i], [evs[i], colv], vs[i])

        start_load(0, xb0, ib0, sb0, sem0)

        @pl.loop(0, NCHUNK, step=2)
        def _(k):
            start_load(k + 1, xb1, ib1, sb1, sem1)
            wait_load(k, xb0, ib0, sb0, sem0)
            process(xb0, ib0, sb0)

            @pl.when(k + 2 < NCHUNK)
            def _():
                start_load(k + 2, xb0, ib0, sb0, sem0)

            wait_load(k + 1, xb1, ib1, sb1, sem1)
            process(xb1, ib1, sb1)

        @pl.loop(0, E)
        def _(e):
            for g in range(CW // L):
                sl = pl.ds(g * L, L)
                parts = [a[e, sl] for a in accs]
                while len(parts) > 1:
                    parts = [parts[i] + parts[i + 1]
                             for i in range(0, len(parts), 2)]
                accs[0][e, sl] = parts[0]

        pltpu.sync_copy(accs[0], out_hbm.at[:, pl.ds(col0, CW)])

    return kern(stateLis, maxpos, sel)


def _tc_matmul_body(w_ref, x_ref, out_ref):
    i = pl.program_id(0)

    @pl.when(i == 0)
    def _():
        out_ref[...] = jnp.zeros_like(out_ref)

    out_ref[...] += lax.dot_general(
        w_ref[...], x_ref[...], (((0,), (0,)), ((), ())),
        preferred_element_type=jnp.float32)


def _tc_matmul(w, stateLis):
    return pl.pallas_call(
        _tc_matmul_body,
        grid=(BTC // BK,),
        in_specs=[
            pl.BlockSpec((BK, E), lambda i: (BSC // BK + i, 0)),
            pl.BlockSpec((BK, D), lambda i: (BSC // BK + i, 0)),
        ],
        out_specs=pl.BlockSpec((E, D), lambda i: (0, 0)),
        out_shape=jax.ShapeDtypeStruct((E, D), jnp.float32),
    )(w, stateLis)


def _combine_body(a_ref, b_ref, den_ref, out_ref):
    den = den_ref[...]
    inv = jnp.where(den > 0.0, 1.0 / den, 1.0)
    r = lax.broadcasted_iota(jnp.int32, (E, E), 0)
    c = lax.broadcasted_iota(jnp.int32, (E, E), 1)
    inv_col = jnp.sum(jnp.where(r == c, jnp.broadcast_to(inv, (E, E)), 0.0),
                      axis=1, keepdims=True)
    out_ref[...] = (a_ref[...] + b_ref[...]) * inv_col


def _combine(num_sc, num_tc, den2):
    return pl.pallas_call(
        _combine_body,
        out_shape=jax.ShapeDtypeStruct((E, D), jnp.float32),
    )(num_sc, num_tc, den2)


@functools.lru_cache(maxsize=1)
def _gumbel_host():
    with jax.ensure_compile_time_eval():
        gkey = jax.random.fold_in(jax.random.key(0), 7)
        return np.asarray(jax.random.gumbel(gkey, (B, E), dtype=jnp.float32))


def kernel(stateLis, probability, LastCliOnSerDic, scalar):
    gumbel = jnp.asarray(_gumbel_host())
    scal = jnp.asarray(scalar, jnp.float32).reshape(1, 1)

    gs, maxpos2, sel2, w, den2 = _routing(probability, gumbel, scal)
    maxpos = maxpos2.reshape(B)
    sel = sel2.reshape(B)

    num_sc = _sc_segment_sum(stateLis, maxpos, sel)
    num_tc = _tc_matmul(w, stateLis)
    agg = _combine(num_sc, num_tc, den2)
    return (gs, agg, maxpos)
